# baseline (speedup 1.0000x reference)
import functools

import jax
import jax.numpy as jnp
from jax import lax
from jax.experimental import pallas as pl
from jax.experimental.pallas import tpu as pltpu
from jax.experimental.pallas import tpu_sc as plsc

_N = 10000
_E = 320000
_DIN = 512
_H = 128
_C = 4

_NC = 2
_NS = 16
_L = 16

_NW = _NC * _NS
_CW = 128
_EROWS = _E // _CW
_RPW = 80
_RPW_LAST = _EROWS - _RPW * (_NW - 1)
_RW = 624
_RREM = _N - _RW * _NS
_ZR = 16

_mesh = plsc.VectorSubcoreMesh(core_axis_name="c", subcore_axis_name="s")


def _fill_const(ref, nrows, vec):
    def row(i, carry):
        for j in range(_H // _L):
            ref[i, pl.ds(j * _L, _L)] = vec
        return carry

    lax.fori_loop(0, nrows, row, 0)


def _zero_acc(acc_sh, zero_v, zsem, s):
    _fill_const(zero_v, _ZR, jnp.zeros((_L,), jnp.float32))
    nz = _RW // _ZR

    def fire(i, carry):
        pltpu.async_copy(zero_v, acc_sh.at[pl.ds(s * _RW + i * _ZR, _ZR)],
                         zsem)
        return carry

    lax.fori_loop(0, nz, fire, 0)

    @pl.when(s == 0)
    def _():
        pltpu.sync_copy(zero_v, acc_sh.at[pl.ds(_RW * _NS, _RREM)])

    def drain(i, carry):
        pltpu.make_async_copy(
            zero_v, acc_sh.at[pl.ds(s * _RW + i * _ZR, _ZR)], zsem).wait()
        return carry

    lax.fori_loop(0, nz, drain, 0)


def _write_out(acc_sh, out_hbm, c, s):
    pltpu.sync_copy(acc_sh.at[pl.ds(s * _RW, _RW)],
                    out_hbm.at[c, pl.ds(s * _RW, _RW)])

    @pl.when(s == 0)
    def _():
        pltpu.sync_copy(acc_sh.at[pl.ds(_RW * _NS, _RREM)],
                        out_hbm.at[c, pl.ds(_RW * _NS, _RREM)])


def _load_idx(idx2_hbm, idx_v, wid):
    @pl.when(wid < _NW - 1)
    def _():
        pltpu.sync_copy(idx2_hbm.at[pl.ds(wid * _RPW, _RPW)], idx_v)

    @pl.when(wid == _NW - 1)
    def _():
        pltpu.sync_copy(idx2_hbm.at[pl.ds((_NW - 1) * _RPW, _RPW_LAST)],
                        idx_v.at[pl.ds(0, _RPW_LAST)])


@functools.partial(
    pl.kernel,
    mesh=_mesh,
    out_type=jax.ShapeDtypeStruct((_NC, _N, _H), jnp.float32),
    scratch_types=[
        pltpu.VMEM((_RPW, _CW), jnp.int32),
        pltpu.VMEM((_CW, _H), jnp.float32),
        pltpu.VMEM((_ZR, _H), jnp.float32),
        pltpu.VMEM_SHARED((_N, _H), jnp.float32),
        pltpu.SemaphoreType.DMA,
        pltpu.SemaphoreType.DMA,
    ],
)
def _deg_kernel(dst2_hbm, out_hbm, dst_i, ones_v, zero_v, acc_sh, ssem, zsem):
    c = lax.axis_index("c")
    s = lax.axis_index("s")
    wid = c * _NS + s
    nch = jnp.where(wid == _NW - 1, _RPW_LAST, _RPW)

    _load_idx(dst2_hbm, dst_i, wid)
    _fill_const(ones_v, _CW, jnp.full((_L,), 1.0, jnp.float32))
    _zero_acc(acc_sh, zero_v, zsem, s)
    plsc.subcore_barrier()

    def fire(j, carry):
        pltpu.async_copy(ones_v, acc_sh.at[dst_i.at[j]], ssem, add=True)
        return carry

    lax.fori_loop(0, nch, fire, 0)

    def drain(j, carry):
        pltpu.make_async_copy(ones_v, acc_sh.at[dst_i.at[j]], ssem).wait()
        return carry

    lax.fori_loop(0, nch, drain, 0)
    plsc.subcore_barrier()
    _write_out(acc_sh, out_hbm, c, s)


@functools.partial(
    pl.kernel,
    mesh=_mesh,
    out_type=jax.ShapeDtypeStruct((_NC, _N, _H), jnp.float32),
    scratch_types=[
        pltpu.VMEM((_RPW, _CW), jnp.int32),
        pltpu.VMEM((_CW,), jnp.int32),
        pltpu.VMEM((_CW,), jnp.int32),
        pltpu.VMEM((_CW, _H), jnp.float32),
        pltpu.VMEM((_CW, _H), jnp.float32),
        pltpu.VMEM((_ZR, _H), jnp.float32),
        pltpu.VMEM_SHARED((_N, _H), jnp.float32),
        pltpu.SemaphoreType.DMA,
        pltpu.SemaphoreType.DMA,
        pltpu.SemaphoreType.DMA,
        pltpu.SemaphoreType.DMA,
        pltpu.SemaphoreType.DMA,
        pltpu.SemaphoreType.DMA,
        pltpu.SemaphoreType.DMA,
    ],
)
def _scatter_kernel(y_hbm, src_hbm, dst2_hbm, out_hbm,
                    dst_i, ib0, ib1, rows0, rows1, zero_v, acc_sh,
                    i0, i1, g0, g1, s0, s1, zsem):
    c = lax.axis_index("c")
    s = lax.axis_index("s")
    wid = c * _NS + s
    nch = jnp.where(wid == _NW - 1, _RPW_LAST, _RPW)
    npair = nch // 2
    eoff = wid * (_RPW * _CW)

    ibs = [ib0, ib1]
    isem = [i0, i1]
    rows = [rows0, rows1]
    gsem = [g0, g1]
    ssem = [s0, s1]

    def idx_load(j, b):
        pltpu.async_copy(src_hbm.at[pl.ds(eoff + j * _CW, _CW)], ibs[b],
                         isem[b])

    def idx_wait(j, b):
        pltpu.make_async_copy(src_hbm.at[pl.ds(eoff + j * _CW, _CW)], ibs[b],
                              isem[b]).wait()

    def gather_start(b):
        pltpu.async_copy(y_hbm.at[ibs[b]], rows[b], gsem[b])

    def gather_wait(b):
        pltpu.make_async_copy(y_hbm.at[ibs[b]], rows[b], gsem[b]).wait()

    def scat_start(j, b):
        pltpu.async_copy(rows[b], acc_sh.at[dst_i.at[j]], ssem[b], add=True)

    def scat_wait(j, b):
        pltpu.make_async_copy(rows[b], acc_sh.at[dst_i.at[j]],
                              ssem[b]).wait()

    _load_idx(dst2_hbm, dst_i, wid)
    idx_load(0, 0)
    idx_load(1, 1)
    idx_wait(0, 0)
    gather_start(0)
    idx_wait(1, 1)
    gather_start(1)
    _zero_acc(acc_sh, zero_v, zsem, s)
    plsc.subcore_barrier()

    def pair(p, carry):
        j0 = 2 * p
        gather_wait(0)
        scat_start(j0, 0)

        @pl.when(p + 1 < npair)
        def _():
            idx_load(j0 + 2, 0)

        gather_wait(1)
        scat_start(j0 + 1, 1)

        @pl.when(p + 1 < npair)
        def _():
            idx_load(j0 + 3, 1)
            scat_wait(j0, 0)
            idx_wait(j0 + 2, 0)
            gather_start(0)
            scat_wait(j0 + 1, 1)
            idx_wait(j0 + 3, 1)
            gather_start(1)

        return carry

    lax.fori_loop(0, npair, pair, 0)
    scat_wait(0, 0)
    scat_wait(1, 1)
    plsc.subcore_barrier()
    _write_out(acc_sh, out_hbm, c, s)


_BR = 2000
_G = _N // _BR


def _dinv_from(deg_ref):
    d = deg_ref[0, :, 0:1] + deg_ref[1, :, 0:1] + 1.0
    return lax.rsqrt(d)


def _tc_fc_body(deg_ref, x_ref, wfc_ref, bfc_ref, w1_ref, y_ref):
    dinv = _dinv_from(deg_ref)
    h = jnp.maximum(
        jnp.dot(x_ref[...], wfc_ref[...], preferred_element_type=jnp.float32)
        + bfc_ref[...], 0.0)
    xw = jnp.dot(h, w1_ref[...], preferred_element_type=jnp.float32)
    y_ref[...] = xw * dinv


def _tc_fc(degp, x, wfc, bfc, w1):
    return pl.pallas_call(
        _tc_fc_body,
        grid=(_G,),
        in_specs=[
            pl.BlockSpec((_NC, _BR, _H), lambda i: (0, i, 0)),
            pl.BlockSpec((_BR, _DIN), lambda i: (i, 0)),
            pl.BlockSpec((_DIN, _H), lambda i: (0, 0)),
            pl.BlockSpec((1, _H), lambda i: (0, 0)),
            pl.BlockSpec((_H, _H), lambda i: (0, 0)),
        ],
        out_specs=pl.BlockSpec((_BR, _H), lambda i: (i, 0)),
        out_shape=jax.ShapeDtypeStruct((_N, _H), jnp.float32),
    )(degp, x, wfc, bfc, w1)


def _tc_mid_body(deg_ref, s_ref, y_ref, b_ref, w_ref, o_ref):
    dinv = _dinv_from(deg_ref)
    x1 = jnp.maximum(
        dinv * (s_ref[0] + s_ref[1] + y_ref[...]) + b_ref[...], 0.0)
    o_ref[...] = jnp.dot(x1, w_ref[...],
                         preferred_element_type=jnp.float32) * dinv


def _tc_mid(degp, s_part, y, b, w):
    return pl.pallas_call(
        _tc_mid_body,
        grid=(_G,),
        in_specs=[
            pl.BlockSpec((_NC, _BR, _H), lambda i: (0, i, 0)),
            pl.BlockSpec((_NC, _BR, _H), lambda i: (0, i, 0)),
            pl.BlockSpec((_BR, _H), lambda i: (i, 0)),
            pl.BlockSpec((1, _H), lambda i: (0, 0)),
            pl.BlockSpec((_H, _H), lambda i: (0, 0)),
        ],
        out_specs=pl.BlockSpec((_BR, _H), lambda i: (i, 0)),
        out_shape=jax.ShapeDtypeStruct((_N, _H), jnp.float32),
    )(degp, s_part, y, b, w)


def _tc_pool_body(deg_ref, s_ref, y_ref, b_ref, v_ref, u_ref, wa_ref,
                  wh_ref, bh_ref, out_ref, num_acc, m_acc, den_acc):
    i = pl.program_id(0)
    dinv = _dinv_from(deg_ref)
    x2 = jnp.maximum(
        dinv * (s_ref[0] + s_ref[1] + y_ref[...]) + b_ref[...], 0.0)
    a = jnp.tanh(jnp.dot(x2, v_ref[...], preferred_element_type=jnp.float32))
    g = jax.nn.sigmoid(
        jnp.dot(x2, u_ref[...], preferred_element_type=jnp.float32))
    t = jnp.dot(a * g, wa_ref[...], preferred_element_type=jnp.float32)

    bm = jnp.max(t)
    m_old = jnp.where(i == 0, -3e38, m_acc[0])
    den_old = jnp.where(i == 0, 0.0, den_acc[0])
    num_old = jnp.where(i == 0, 0.0, num_acc[0:1, :])
    m_new = jnp.maximum(m_old, bm)
    alpha = jnp.exp(m_old - m_new)
    w = jnp.exp(t - m_new)
    num_new = num_old * alpha + lax.dot_general(
        w, x2, (((0,), (0,)), ((), ())), preferred_element_type=jnp.float32)
    den_new = den_old * alpha + jnp.sum(w)
    m_acc[0] = m_new
    den_acc[0] = den_new
    num_acc[0:1, :] = num_new
    out_ref[...] = (jnp.dot(num_new / den_new, wh_ref[...],
                            preferred_element_type=jnp.float32) + bh_ref[...])


def _tc_pool(degp, s_part, y, b, v, u, wa, wh, bh):
    return pl.pallas_call(
        _tc_pool_body,
        grid=(_G,),
        in_specs=[
            pl.BlockSpec((_NC, _BR, _H), lambda i: (0, i, 0)),
            pl.BlockSpec((_NC, _BR, _H), lambda i: (0, i, 0)),
            pl.BlockSpec((_BR, _H), lambda i: (i, 0)),
            pl.BlockSpec((1, _H), lambda i: (0, 0)),
            pl.BlockSpec((_H, _H), lambda i: (0, 0)),
            pl.BlockSpec((_H, _H), lambda i: (0, 0)),
            pl.BlockSpec((_H, 1), lambda i: (0, 0)),
            pl.BlockSpec((_H, _C), lambda i: (0, 0)),
            pl.BlockSpec((1, _C), lambda i: (0, 0)),
        ],
        out_specs=pl.BlockSpec((1, _C), lambda i: (0, 0)),
        out_shape=jax.ShapeDtypeStruct((1, _C), jnp.float32),
        scratch_shapes=[
            pltpu.VMEM((8, _H), jnp.float32),
            pltpu.SMEM((1,), jnp.float32),
            pltpu.SMEM((1,), jnp.float32),
        ],
    )(degp, s_part, y, b, v, u, wa, wh, bh)


def kernel(x, edge_index, W_fc, b_fc, W1, b1, W2, b2, V, U, w_attn,
           W_head, b_head):
    src = edge_index[0]
    dst2 = edge_index[1].reshape(_EROWS, _CW)
    degp = _deg_kernel(dst2)
    y1 = _tc_fc(degp, x, W_fc, b_fc.reshape(1, _H), W1)
    s1 = _scatter_kernel(y1, src, dst2)
    y2 = _tc_mid(degp, s1, y1, b1.reshape(1, _H), W2)
    s2 = _scatter_kernel(y2, src, dst2)
    out = _tc_pool(degp, s2, y2, b2.reshape(1, _H), V, U, w_attn,
                   W_head, b_head.reshape(1, _C))
    return out


# speedup vs baseline: 1.1704x; 1.1704x over previous
import functools

import jax
import jax.numpy as jnp
from jax import lax
from jax.experimental import pallas as pl
from jax.experimental.pallas import tpu as pltpu
from jax.experimental.pallas import tpu_sc as plsc

_N = 10000
_E = 320000
_DIN = 512
_H = 128
_C = 4

_NC = 2
_NS = 16
_L = 16

_NW = _NC * _NS
_CW = 128
_EROWS = _E // _CW
_RPW = 80
_RPW_LAST = _EROWS - _RPW * (_NW - 1)
_RW = 624
_RREM = _N - _RW * _NS
_ZR = 8

_mesh = plsc.VectorSubcoreMesh(core_axis_name="c", subcore_axis_name="s")


def _fill_const(ref, nrows, vec):
    def row(i, carry):
        for j in range(_H // _L):
            ref[i, pl.ds(j * _L, _L)] = vec
        return carry

    lax.fori_loop(0, nrows, row, 0)


def _zero_acc(acc_sh, zero_v, zsem, s):
    _fill_const(zero_v, _ZR, jnp.zeros((_L,), jnp.float32))
    nz = _RW // _ZR

    def fire(i, carry):
        pltpu.async_copy(zero_v, acc_sh.at[pl.ds(s * _RW + i * _ZR, _ZR)],
                         zsem)
        return carry

    lax.fori_loop(0, nz, fire, 0)

    @pl.when(s == 0)
    def _():
        pltpu.sync_copy(zero_v, acc_sh.at[pl.ds(_RW * _NS, _ZR)])
        pltpu.sync_copy(zero_v, acc_sh.at[pl.ds(_RW * _NS + _ZR, _ZR)])

    def drain(i, carry):
        pltpu.make_async_copy(
            zero_v, acc_sh.at[pl.ds(s * _RW + i * _ZR, _ZR)], zsem).wait()
        return carry

    lax.fori_loop(0, nz, drain, 0)


def _write_out(acc_sh, out_hbm, c, s):
    pltpu.sync_copy(acc_sh.at[pl.ds(s * _RW, _RW)],
                    out_hbm.at[c, pl.ds(s * _RW, _RW)])

    @pl.when(s == 0)
    def _():
        pltpu.sync_copy(acc_sh.at[pl.ds(_RW * _NS, _RREM)],
                        out_hbm.at[c, pl.ds(_RW * _NS, _RREM)])


def _load_idx(idx2_hbm, idx_v, wid):
    @pl.when(wid < _NW - 1)
    def _():
        pltpu.sync_copy(idx2_hbm.at[pl.ds(wid * _RPW, _RPW)], idx_v)

    @pl.when(wid == _NW - 1)
    def _():
        pltpu.sync_copy(idx2_hbm.at[pl.ds((_NW - 1) * _RPW, _RPW_LAST)],
                        idx_v.at[pl.ds(0, _RPW_LAST)])


@functools.partial(
    pl.kernel,
    mesh=_mesh,
    out_type=jax.ShapeDtypeStruct((_NC, _N, _H), jnp.float32),
    scratch_types=[
        pltpu.VMEM((_RPW, _CW), jnp.int32),
        pltpu.VMEM((_CW, _H), jnp.float32),
        pltpu.VMEM((_ZR, _H), jnp.float32),
        pltpu.VMEM_SHARED((_N, _H), jnp.float32),
        pltpu.SemaphoreType.DMA,
        pltpu.SemaphoreType.DMA,
    ],
)
def _deg_kernel(dst2_hbm, out_hbm, dst_i, ones_v, zero_v, acc_sh, ssem, zsem):
    c = lax.axis_index("c")
    s = lax.axis_index("s")
    wid = c * _NS + s
    nch = jnp.where(wid == _NW - 1, _RPW_LAST, _RPW)

    _load_idx(dst2_hbm, dst_i, wid)
    _fill_const(ones_v, _CW, jnp.full((_L,), 1.0, jnp.float32))
    _zero_acc(acc_sh, zero_v, zsem, s)
    plsc.subcore_barrier()

    def fire(j, carry):
        pltpu.async_copy(ones_v, acc_sh.at[dst_i.at[j]], ssem, add=True)
        return carry

    lax.fori_loop(0, nch, fire, 0)

    def drain(j, carry):
        pltpu.make_async_copy(ones_v, acc_sh.at[dst_i.at[j]], ssem).wait()
        return carry

    lax.fori_loop(0, nch, drain, 0)
    plsc.subcore_barrier()
    _write_out(acc_sh, out_hbm, c, s)


@functools.partial(
    pl.kernel,
    mesh=_mesh,
    out_type=jax.ShapeDtypeStruct((_NC, _N, _H), jnp.float32),
    scratch_types=[
        pltpu.VMEM((_CW,), jnp.int32),
        pltpu.VMEM((_CW,), jnp.int32),
        pltpu.VMEM((_CW,), jnp.int32),
        pltpu.VMEM((_CW,), jnp.int32),
        pltpu.VMEM((_CW,), jnp.int32),
        pltpu.VMEM((_CW,), jnp.int32),
        pltpu.VMEM((_CW, _H), jnp.float32),
        pltpu.VMEM((_CW, _H), jnp.float32),
        pltpu.VMEM((_CW, _H), jnp.float32),
        pltpu.VMEM((_ZR, _H), jnp.float32),
        pltpu.VMEM_SHARED((_N, _H), jnp.float32),
        pltpu.SemaphoreType.DMA,
        pltpu.SemaphoreType.DMA,
        pltpu.SemaphoreType.DMA,
        pltpu.SemaphoreType.DMA,
        pltpu.SemaphoreType.DMA,
        pltpu.SemaphoreType.DMA,
        pltpu.SemaphoreType.DMA,
        pltpu.SemaphoreType.DMA,
        pltpu.SemaphoreType.DMA,
        pltpu.SemaphoreType.DMA,
    ],
)
def _scatter_kernel(y_hbm, src_hbm, dst_hbm, out_hbm,
                    ib0, ib1, ib2, db0, db1, db2, rows0, rows1, rows2,
                    zero_v, acc_sh,
                    is0, is1, is2, id0, id1, id2, g0, g1, g2, zsem):
    c = lax.axis_index("c")
    s = lax.axis_index("s")
    wid = c * _NS + s
    nch = jnp.where(wid == _NW - 1, _RPW_LAST, _RPW)
    eoff = wid * (_RPW * _CW)

    ibs = [ib0, ib1, ib2]
    dbs = [db0, db1, db2]
    isem = [is0, is1, is2]
    dsem = [id0, id1, id2]
    rows = [rows0, rows1, rows2]
    gsem = [g0, g1, g2]

    def src_load(j, b):
        pltpu.async_copy(src_hbm.at[pl.ds(eoff + j * _CW, _CW)], ibs[b],
                         isem[b])

    def src_wait(j, b):
        pltpu.make_async_copy(src_hbm.at[pl.ds(eoff + j * _CW, _CW)], ibs[b],
                              isem[b]).wait()

    def dst_load(j, b):
        pltpu.async_copy(dst_hbm.at[pl.ds(eoff + j * _CW, _CW)], dbs[b],
                         dsem[b])

    def dst_wait(j, b):
        pltpu.make_async_copy(dst_hbm.at[pl.ds(eoff + j * _CW, _CW)], dbs[b],
                              dsem[b]).wait()

    def gather_start(b):
        pltpu.async_copy(y_hbm.at[ibs[b]], rows[b], gsem[b])

    def gather_wait(b):
        pltpu.make_async_copy(y_hbm.at[ibs[b]], rows[b], gsem[b]).wait()

    for b in range(3):
        src_load(b, b)
        dst_load(b, b)
    src_wait(0, 0)
    gather_start(0)
    src_wait(1, 1)
    gather_start(1)
    _zero_acc(acc_sh, zero_v, zsem, s)
    plsc.subcore_barrier()

    ntrip = (nch + 2) // 3

    def trip(t, carry):
        for b in range(3):
            j = 3 * t + b

            @pl.when(j < nch)
            def _():
                gather_wait(b)

                @pl.when(j + 2 < nch)
                def _():
                    b2 = (b + 2) % 3
                    src_wait(j + 2, b2)
                    gather_start(b2)

                dst_wait(j, b)
                pltpu.sync_copy(rows[b], acc_sh.at[dbs[b]], add=True)

                @pl.when(j + 3 < nch)
                def _():
                    src_load(j + 3, b)
                    dst_load(j + 3, b)

        return carry

    lax.fori_loop(0, ntrip, trip, 0)
    plsc.subcore_barrier()
    _write_out(acc_sh, out_hbm, c, s)


_BR = 2000
_G = _N // _BR


def _dinv_from(deg_ref):
    d = deg_ref[0, :, 0:1] + deg_ref[1, :, 0:1] + 1.0
    return lax.rsqrt(d)


def _tc_fc_body(deg_ref, x_ref, wfc_ref, bfc_ref, w1_ref, y_ref):
    dinv = _dinv_from(deg_ref)
    h = jnp.maximum(
        jnp.dot(x_ref[...], wfc_ref[...], preferred_element_type=jnp.float32)
        + bfc_ref[...], 0.0)
    xw = jnp.dot(h, w1_ref[...], preferred_element_type=jnp.float32)
    y_ref[...] = xw * dinv


def _tc_fc(degp, x, wfc, bfc, w1):
    return pl.pallas_call(
        _tc_fc_body,
        grid=(_G,),
        in_specs=[
            pl.BlockSpec((_NC, _BR, _H), lambda i: (0, i, 0)),
            pl.BlockSpec((_BR, _DIN), lambda i: (i, 0)),
            pl.BlockSpec((_DIN, _H), lambda i: (0, 0)),
            pl.BlockSpec((1, _H), lambda i: (0, 0)),
            pl.BlockSpec((_H, _H), lambda i: (0, 0)),
        ],
        out_specs=pl.BlockSpec((_BR, _H), lambda i: (i, 0)),
        out_shape=jax.ShapeDtypeStruct((_N, _H), jnp.float32),
    )(degp, x, wfc, bfc, w1)


def _tc_mid_body(deg_ref, s_ref, y_ref, b_ref, w_ref, o_ref):
    dinv = _dinv_from(deg_ref)
    x1 = jnp.maximum(
        dinv * (s_ref[0] + s_ref[1] + y_ref[...]) + b_ref[...], 0.0)
    o_ref[...] = jnp.dot(x1, w_ref[...],
                         preferred_element_type=jnp.float32) * dinv


def _tc_mid(degp, s_part, y, b, w):
    return pl.pallas_call(
        _tc_mid_body,
        grid=(_G,),
        in_specs=[
            pl.BlockSpec((_NC, _BR, _H), lambda i: (0, i, 0)),
            pl.BlockSpec((_NC, _BR, _H), lambda i: (0, i, 0)),
            pl.BlockSpec((_BR, _H), lambda i: (i, 0)),
            pl.BlockSpec((1, _H), lambda i: (0, 0)),
            pl.BlockSpec((_H, _H), lambda i: (0, 0)),
        ],
        out_specs=pl.BlockSpec((_BR, _H), lambda i: (i, 0)),
        out_shape=jax.ShapeDtypeStruct((_N, _H), jnp.float32),
    )(degp, s_part, y, b, w)


def _tc_pool_body(deg_ref, s_ref, y_ref, b_ref, v_ref, u_ref, wa_ref,
                  wh_ref, bh_ref, out_ref, num_acc, m_acc, den_acc):
    i = pl.program_id(0)
    dinv = _dinv_from(deg_ref)
    x2 = jnp.maximum(
        dinv * (s_ref[0] + s_ref[1] + y_ref[...]) + b_ref[...], 0.0)
    a = jnp.tanh(jnp.dot(x2, v_ref[...], preferred_element_type=jnp.float32))
    g = jax.nn.sigmoid(
        jnp.dot(x2, u_ref[...], preferred_element_type=jnp.float32))
    t = jnp.dot(a * g, wa_ref[...], preferred_element_type=jnp.float32)

    bm = jnp.max(t)
    m_old = jnp.where(i == 0, -3e38, m_acc[0])
    den_old = jnp.where(i == 0, 0.0, den_acc[0])
    num_old = jnp.where(i == 0, 0.0, num_acc[0:1, :])
    m_new = jnp.maximum(m_old, bm)
    alpha = jnp.exp(m_old - m_new)
    w = jnp.exp(t - m_new)
    num_new = num_old * alpha + lax.dot_general(
        w, x2, (((0,), (0,)), ((), ())), preferred_element_type=jnp.float32)
    den_new = den_old * alpha + jnp.sum(w)
    m_acc[0] = m_new
    den_acc[0] = den_new
    num_acc[0:1, :] = num_new
    out_ref[...] = (jnp.dot(num_new / den_new, wh_ref[...],
                            preferred_element_type=jnp.float32) + bh_ref[...])


def _tc_pool(degp, s_part, y, b, v, u, wa, wh, bh):
    return pl.pallas_call(
        _tc_pool_body,
        grid=(_G,),
        in_specs=[
            pl.BlockSpec((_NC, _BR, _H), lambda i: (0, i, 0)),
            pl.BlockSpec((_NC, _BR, _H), lambda i: (0, i, 0)),
            pl.BlockSpec((_BR, _H), lambda i: (i, 0)),
            pl.BlockSpec((1, _H), lambda i: (0, 0)),
            pl.BlockSpec((_H, _H), lambda i: (0, 0)),
            pl.BlockSpec((_H, _H), lambda i: (0, 0)),
            pl.BlockSpec((_H, 1), lambda i: (0, 0)),
            pl.BlockSpec((_H, _C), lambda i: (0, 0)),
            pl.BlockSpec((1, _C), lambda i: (0, 0)),
        ],
        out_specs=pl.BlockSpec((1, _C), lambda i: (0, 0)),
        out_shape=jax.ShapeDtypeStruct((1, _C), jnp.float32),
        scratch_shapes=[
            pltpu.VMEM((8, _H), jnp.float32),
            pltpu.SMEM((1,), jnp.float32),
            pltpu.SMEM((1,), jnp.float32),
        ],
    )(degp, s_part, y, b, v, u, wa, wh, bh)


def kernel(x, edge_index, W_fc, b_fc, W1, b1, W2, b2, V, U, w_attn,
           W_head, b_head):
    src = edge_index[0]
    dst = edge_index[1]
    dst2 = dst.reshape(_EROWS, _CW)
    degp = _deg_kernel(dst2)
    y1 = _tc_fc(degp, x, W_fc, b_fc.reshape(1, _H), W1)
    s1 = _scatter_kernel(y1, src, dst)
    y2 = _tc_mid(degp, s1, y1, b1.reshape(1, _H), W2)
    s2 = _scatter_kernel(y2, src, dst)
    out = _tc_pool(degp, s2, y2, b2.reshape(1, _H), V, U, w_attn,
                   W_head, b_head.reshape(1, _C))
    return out


# speedup vs baseline: 1.1756x; 1.0045x over previous
import functools

import jax
import jax.numpy as jnp
from jax import lax
from jax.experimental import pallas as pl
from jax.experimental.pallas import tpu as pltpu
from jax.experimental.pallas import tpu_sc as plsc

_N = 10000
_E = 320000
_DIN = 512
_H = 128
_C = 4

_NC = 2
_NS = 16
_L = 16

_NW = _NC * _NS
_CW = 128
_EROWS = _E // _CW
_RPW = 80
_RPW_LAST = _EROWS - _RPW * (_NW - 1)
_RW = 624
_RREM = _N - _RW * _NS
_ZR = 8

_mesh = plsc.VectorSubcoreMesh(core_axis_name="c", subcore_axis_name="s")


def _fill_const(ref, nrows, vec):
    def row(i, carry):
        for j in range(_H // _L):
            ref[i, pl.ds(j * _L, _L)] = vec
        return carry

    lax.fori_loop(0, nrows, row, 0)


def _zero_acc(acc_sh, zero_v, zsem, s):
    _fill_const(zero_v, _ZR, jnp.zeros((_L,), jnp.float32))
    nz = _RW // _ZR

    def fire(i, carry):
        pltpu.async_copy(zero_v, acc_sh.at[pl.ds(s * _RW + i * _ZR, _ZR)],
                         zsem)
        return carry

    lax.fori_loop(0, nz, fire, 0)

    @pl.when(s == 0)
    def _():
        pltpu.sync_copy(zero_v, acc_sh.at[pl.ds(_RW * _NS, _ZR)])
        pltpu.sync_copy(zero_v, acc_sh.at[pl.ds(_RW * _NS + _ZR, _ZR)])

    def drain(i, carry):
        pltpu.make_async_copy(
            zero_v, acc_sh.at[pl.ds(s * _RW + i * _ZR, _ZR)], zsem).wait()
        return carry

    lax.fori_loop(0, nz, drain, 0)


def _write_out(acc_sh, out_hbm, c, s):
    pltpu.sync_copy(acc_sh.at[pl.ds(s * _RW, _RW)],
                    out_hbm.at[c, pl.ds(s * _RW, _RW)])

    @pl.when(s == 0)
    def _():
        pltpu.sync_copy(acc_sh.at[pl.ds(_RW * _NS, _RREM)],
                        out_hbm.at[c, pl.ds(_RW * _NS, _RREM)])


def _load_idx(idx2_hbm, idx_v, wid):
    @pl.when(wid < _NW - 1)
    def _():
        pltpu.sync_copy(idx2_hbm.at[pl.ds(wid * _RPW, _RPW)], idx_v)

    @pl.when(wid == _NW - 1)
    def _():
        pltpu.sync_copy(idx2_hbm.at[pl.ds((_NW - 1) * _RPW, _RPW_LAST)],
                        idx_v.at[pl.ds(0, _RPW_LAST)])


@functools.partial(
    pl.kernel,
    mesh=_mesh,
    out_type=jax.ShapeDtypeStruct((_NC, _N, _H), jnp.float32),
    scratch_types=[
        pltpu.VMEM((_RPW, _CW), jnp.int32),
        pltpu.VMEM((_CW, _H), jnp.float32),
        pltpu.VMEM((_ZR, _H), jnp.float32),
        pltpu.VMEM_SHARED((_N, _H), jnp.float32),
        pltpu.SemaphoreType.DMA,
        pltpu.SemaphoreType.DMA,
    ],
)
def _deg_kernel(dst2_hbm, out_hbm, dst_i, ones_v, zero_v, acc_sh, ssem, zsem):
    c = lax.axis_index("c")
    s = lax.axis_index("s")
    wid = c * _NS + s
    nch = jnp.where(wid == _NW - 1, _RPW_LAST, _RPW)

    _load_idx(dst2_hbm, dst_i, wid)
    _fill_const(ones_v, _CW, jnp.full((_L,), 1.0, jnp.float32))
    _zero_acc(acc_sh, zero_v, zsem, s)
    plsc.subcore_barrier()

    def fire(j, carry):
        pltpu.async_copy(ones_v, acc_sh.at[dst_i.at[j]], ssem, add=True)
        return carry

    lax.fori_loop(0, nch, fire, 0)

    def drain(j, carry):
        pltpu.make_async_copy(ones_v, acc_sh.at[dst_i.at[j]], ssem).wait()
        return carry

    lax.fori_loop(0, nch, drain, 0)
    plsc.subcore_barrier()
    _write_out(acc_sh, out_hbm, c, s)


@functools.partial(
    pl.kernel,
    mesh=_mesh,
    out_type=jax.ShapeDtypeStruct((_NC, _N, _H), jnp.float32),
    scratch_types=[
        pltpu.VMEM((_CW,), jnp.int32),
        pltpu.VMEM((_CW,), jnp.int32),
        pltpu.VMEM((_CW,), jnp.int32),
        pltpu.VMEM((_CW,), jnp.int32),
        pltpu.VMEM((_CW,), jnp.int32),
        pltpu.VMEM((_CW,), jnp.int32),
        pltpu.VMEM((_CW, _H), jnp.float32),
        pltpu.VMEM((_CW, _H), jnp.float32),
        pltpu.VMEM((_CW, _H), jnp.float32),
        pltpu.VMEM((_ZR, _H), jnp.float32),
        pltpu.VMEM_SHARED((_N, _H), jnp.float32),
        pltpu.SemaphoreType.DMA,
        pltpu.SemaphoreType.DMA,
        pltpu.SemaphoreType.DMA,
        pltpu.SemaphoreType.DMA,
        pltpu.SemaphoreType.DMA,
        pltpu.SemaphoreType.DMA,
        pltpu.SemaphoreType.DMA,
        pltpu.SemaphoreType.DMA,
        pltpu.SemaphoreType.DMA,
        pltpu.SemaphoreType.DMA,
    ],
)
def _scatter_kernel(y_hbm, src_hbm, dst_hbm, out_hbm,
                    ib0, ib1, ib2, db0, db1, db2, rows0, rows1, rows2,
                    zero_v, acc_sh,
                    is0, is1, is2, id0, id1, id2, g0, g1, g2, zsem):
    c = lax.axis_index("c")
    s = lax.axis_index("s")
    wid = c * _NS + s
    nch = jnp.where(wid == _NW - 1, _RPW_LAST, _RPW)
    eoff = wid * (_RPW * _CW)

    ibs = [ib0, ib1, ib2]
    dbs = [db0, db1, db2]
    isem = [is0, is1, is2]
    dsem = [id0, id1, id2]
    rows = [rows0, rows1, rows2]
    gsem = [g0, g1, g2]

    def src_load(j, b):
        pltpu.async_copy(src_hbm.at[pl.ds(eoff + j * _CW, _CW)], ibs[b],
                         isem[b])

    def src_wait(j, b):
        pltpu.make_async_copy(src_hbm.at[pl.ds(eoff + j * _CW, _CW)], ibs[b],
                              isem[b]).wait()

    def dst_load(j, b):
        pltpu.async_copy(dst_hbm.at[pl.ds(eoff + j * _CW, _CW)], dbs[b],
                         dsem[b])

    def dst_wait(j, b):
        pltpu.make_async_copy(dst_hbm.at[pl.ds(eoff + j * _CW, _CW)], dbs[b],
                              dsem[b]).wait()

    def gather_start(b):
        pltpu.async_copy(y_hbm.at[ibs[b]], rows[b], gsem[b])

    def gather_wait(b):
        pltpu.make_async_copy(y_hbm.at[ibs[b]], rows[b], gsem[b]).wait()

    for b in range(3):
        src_load(b, b)
        dst_load(b, b)
    src_wait(0, 0)
    gather_start(0)
    src_wait(1, 1)
    gather_start(1)
    _zero_acc(acc_sh, zero_v, zsem, s)
    plsc.subcore_barrier()

    ntrip = (nch + 2) // 3

    def trip(t, carry):
        for b in range(3):
            j = 3 * t + b

            @pl.when(j < nch)
            def _():
                gather_wait(b)

                @pl.when(j + 2 < nch)
                def _():
                    b2 = (b + 2) % 3
                    src_wait(j + 2, b2)
                    gather_start(b2)

                dst_wait(j, b)
                pltpu.sync_copy(rows[b], acc_sh.at[dbs[b]], add=True)

                @pl.when(j + 3 < nch)
                def _():
                    src_load(j + 3, b)
                    dst_load(j + 3, b)

        return carry

    lax.fori_loop(0, ntrip, trip, 0)
    plsc.subcore_barrier()
    _write_out(acc_sh, out_hbm, c, s)


_BR = 2000
_G = _N // _BR


def _dinv_from(deg_ref):
    d = deg_ref[0, :, 0:1] + deg_ref[1, :, 0:1] + 1.0
    return lax.rsqrt(d)


def _tc_fc_body(deg_ref, x_ref, wfc_ref, bfc_ref, w1_ref, y_ref, dinv_ref):
    dinv = _dinv_from(deg_ref)
    h = jnp.maximum(
        jnp.dot(x_ref[...], wfc_ref[...], preferred_element_type=jnp.float32)
        + bfc_ref[...], 0.0)
    xw = jnp.dot(h, w1_ref[...], preferred_element_type=jnp.float32)
    y_ref[...] = xw * dinv
    dinv_ref[...] = dinv


def _tc_fc(degp, x, wfc, bfc, w1):
    return pl.pallas_call(
        _tc_fc_body,
        grid=(_G,),
        in_specs=[
            pl.BlockSpec((_NC, _BR, _H), lambda i: (0, i, 0)),
            pl.BlockSpec((_BR, _DIN), lambda i: (i, 0)),
            pl.BlockSpec((_DIN, _H), lambda i: (0, 0)),
            pl.BlockSpec((1, _H), lambda i: (0, 0)),
            pl.BlockSpec((_H, _H), lambda i: (0, 0)),
        ],
        out_specs=[pl.BlockSpec((_BR, _H), lambda i: (i, 0)),
                   pl.BlockSpec((_BR, 1), lambda i: (i, 0))],
        out_shape=[jax.ShapeDtypeStruct((_N, _H), jnp.float32),
                   jax.ShapeDtypeStruct((_N, 1), jnp.float32)],
    )(degp, x, wfc, bfc, w1)


def _tc_mid_body(dinv_ref, s_ref, y_ref, b_ref, w_ref, o_ref):
    dinv = dinv_ref[...]
    x1 = jnp.maximum(
        dinv * (s_ref[0] + s_ref[1] + y_ref[...]) + b_ref[...], 0.0)
    o_ref[...] = jnp.dot(x1, w_ref[...],
                         preferred_element_type=jnp.float32) * dinv


def _tc_mid(dinv, s_part, y, b, w):
    return pl.pallas_call(
        _tc_mid_body,
        grid=(_G,),
        in_specs=[
            pl.BlockSpec((_BR, 1), lambda i: (i, 0)),
            pl.BlockSpec((_NC, _BR, _H), lambda i: (0, i, 0)),
            pl.BlockSpec((_BR, _H), lambda i: (i, 0)),
            pl.BlockSpec((1, _H), lambda i: (0, 0)),
            pl.BlockSpec((_H, _H), lambda i: (0, 0)),
        ],
        out_specs=pl.BlockSpec((_BR, _H), lambda i: (i, 0)),
        out_shape=jax.ShapeDtypeStruct((_N, _H), jnp.float32),
    )(dinv, s_part, y, b, w)


def _tc_pool_body(dinv_ref, s_ref, y_ref, b_ref, v_ref, u_ref, wa_ref,
                  wh_ref, bh_ref, out_ref, num_acc, m_acc, den_acc):
    i = pl.program_id(0)
    dinv = dinv_ref[...]
    x2 = jnp.maximum(
        dinv * (s_ref[0] + s_ref[1] + y_ref[...]) + b_ref[...], 0.0)
    a = jnp.tanh(jnp.dot(x2, v_ref[...], preferred_element_type=jnp.float32))
    g = jax.nn.sigmoid(
        jnp.dot(x2, u_ref[...], preferred_element_type=jnp.float32))
    t = jnp.dot(a * g, wa_ref[...], preferred_element_type=jnp.float32)

    bm = jnp.max(t)
    m_old = jnp.where(i == 0, -3e38, m_acc[0])
    den_old = jnp.where(i == 0, 0.0, den_acc[0])
    num_old = jnp.where(i == 0, 0.0, num_acc[0:1, :])
    m_new = jnp.maximum(m_old, bm)
    alpha = jnp.exp(m_old - m_new)
    w = jnp.exp(t - m_new)
    num_new = num_old * alpha + lax.dot_general(
        w, x2, (((0,), (0,)), ((), ())), preferred_element_type=jnp.float32)
    den_new = den_old * alpha + jnp.sum(w)
    m_acc[0] = m_new
    den_acc[0] = den_new
    num_acc[0:1, :] = num_new
    out_ref[...] = (jnp.dot(num_new / den_new, wh_ref[...],
                            preferred_element_type=jnp.float32) + bh_ref[...])


def _tc_pool(dinv, s_part, y, b, v, u, wa, wh, bh):
    return pl.pallas_call(
        _tc_pool_body,
        grid=(_G,),
        in_specs=[
            pl.BlockSpec((_BR, 1), lambda i: (i, 0)),
            pl.BlockSpec((_NC, _BR, _H), lambda i: (0, i, 0)),
            pl.BlockSpec((_BR, _H), lambda i: (i, 0)),
            pl.BlockSpec((1, _H), lambda i: (0, 0)),
            pl.BlockSpec((_H, _H), lambda i: (0, 0)),
            pl.BlockSpec((_H, _H), lambda i: (0, 0)),
            pl.BlockSpec((_H, 1), lambda i: (0, 0)),
            pl.BlockSpec((_H, _C), lambda i: (0, 0)),
            pl.BlockSpec((1, _C), lambda i: (0, 0)),
        ],
        out_specs=pl.BlockSpec((1, _C), lambda i: (0, 0)),
        out_shape=jax.ShapeDtypeStruct((1, _C), jnp.float32),
        scratch_shapes=[
            pltpu.VMEM((8, _H), jnp.float32),
            pltpu.SMEM((1,), jnp.float32),
            pltpu.SMEM((1,), jnp.float32),
        ],
    )(dinv, s_part, y, b, v, u, wa, wh, bh)


def kernel(x, edge_index, W_fc, b_fc, W1, b1, W2, b2, V, U, w_attn,
           W_head, b_head):
    src = edge_index[0]
    dst = edge_index[1]
    dst2 = dst.reshape(_EROWS, _CW)
    degp = _deg_kernel(dst2)
    y1, dinv = _tc_fc(degp, x, W_fc, b_fc.reshape(1, _H), W1)
    s1 = _scatter_kernel(y1, src, dst)
    y2 = _tc_mid(dinv, s1, y1, b1.reshape(1, _H), W2)
    s2 = _scatter_kernel(y2, src, dst)
    out = _tc_pool(dinv, s2, y2, b2.reshape(1, _H), V, U, w_attn,
                   W_head, b_head.reshape(1, _C))
    return out


# speedup vs baseline: 1.3259x; 1.1278x over previous
import functools

import jax
import jax.numpy as jnp
from jax import lax
from jax.experimental import pallas as pl
from jax.experimental.pallas import tpu as pltpu
from jax.experimental.pallas import tpu_sc as plsc

_N = 10000
_E = 320000
_DIN = 512
_H = 128
_C = 4

_NC = 2
_NS = 16
_L = 16

_NW = _NC * _NS
_CW = 128
_EROWS = _E // _CW
_RPW = 80
_RPW_LAST = _EROWS - _RPW * (_NW - 1)
_RW = 624
_RREM = _N - _RW * _NS
_ZR = 8

_mesh = plsc.VectorSubcoreMesh(core_axis_name="c", subcore_axis_name="s")


def _fill_const(ref, nrows, vec):
    def row(i, carry):
        for j in range(_H // _L):
            ref[i, pl.ds(j * _L, _L)] = vec
        return carry

    lax.fori_loop(0, nrows, row, 0)


def _zero_acc(acc_sh, zbuf, zsem, s):
    _fill_const(zbuf, _ZR, jnp.zeros((_L,), jnp.float32))
    zero_v = zbuf.at[pl.ds(0, _ZR)]
    nz = _RW // _ZR

    def fire(i, carry):
        pltpu.async_copy(zero_v, acc_sh.at[pl.ds(s * _RW + i * _ZR, _ZR)],
                         zsem)
        return carry

    lax.fori_loop(0, nz, fire, 0)

    @pl.when(s == 0)
    def _():
        pltpu.sync_copy(zero_v, acc_sh.at[pl.ds(_RW * _NS, _ZR)])
        pltpu.sync_copy(zero_v, acc_sh.at[pl.ds(_RW * _NS + _ZR, _ZR)])

    def drain(i, carry):
        pltpu.make_async_copy(
            zero_v, acc_sh.at[pl.ds(s * _RW + i * _ZR, _ZR)], zsem).wait()
        return carry

    lax.fori_loop(0, nz, drain, 0)


def _write_out(acc_sh, out_hbm, c, s):
    pltpu.sync_copy(acc_sh.at[pl.ds(s * _RW, _RW)],
                    out_hbm.at[c, pl.ds(s * _RW, _RW)])

    @pl.when(s == 0)
    def _():
        pltpu.sync_copy(acc_sh.at[pl.ds(_RW * _NS, _RREM)],
                        out_hbm.at[c, pl.ds(_RW * _NS, _RREM)])


def _load_idx(idx2_hbm, idx_v, wid):
    @pl.when(wid < _NW - 1)
    def _():
        pltpu.sync_copy(idx2_hbm.at[pl.ds(wid * _RPW, _RPW)], idx_v)

    @pl.when(wid == _NW - 1)
    def _():
        pltpu.sync_copy(idx2_hbm.at[pl.ds((_NW - 1) * _RPW, _RPW_LAST)],
                        idx_v.at[pl.ds(0, _RPW_LAST)])


@functools.partial(
    pl.kernel,
    mesh=_mesh,
    out_type=jax.ShapeDtypeStruct((_NC, _N, _H), jnp.float32),
    scratch_types=[
        pltpu.VMEM((_RPW, _CW), jnp.int32),
        pltpu.VMEM((_CW, _H), jnp.float32),
        pltpu.VMEM((_ZR, _H), jnp.float32),
        pltpu.VMEM_SHARED((_N, _H), jnp.float32),
        pltpu.SemaphoreType.DMA,
        pltpu.SemaphoreType.DMA,
    ],
)
def _deg_kernel(dst2_hbm, out_hbm, dst_i, ones_v, zero_v, acc_sh, ssem, zsem):
    c = lax.axis_index("c")
    s = lax.axis_index("s")
    wid = c * _NS + s
    nch = jnp.where(wid == _NW - 1, _RPW_LAST, _RPW)

    _load_idx(dst2_hbm, dst_i, wid)
    _fill_const(ones_v, _CW, jnp.full((_L,), 1.0, jnp.float32))
    _zero_acc(acc_sh, zero_v, zsem, s)
    plsc.subcore_barrier()

    def fire(j, carry):
        pltpu.async_copy(ones_v, acc_sh.at[dst_i.at[j]], ssem, add=True)
        return carry

    lax.fori_loop(0, nch, fire, 0)

    def drain(j, carry):
        pltpu.make_async_copy(ones_v, acc_sh.at[dst_i.at[j]], ssem).wait()
        return carry

    lax.fori_loop(0, nch, drain, 0)
    plsc.subcore_barrier()
    _write_out(acc_sh, out_hbm, c, s)


@functools.partial(
    pl.kernel,
    mesh=_mesh,
    out_type=jax.ShapeDtypeStruct((_NC, _N, _H), jnp.float32),
    scratch_types=[
        pltpu.VMEM((_CW,), jnp.int32),
        pltpu.VMEM((_CW,), jnp.int32),
        pltpu.VMEM((_CW,), jnp.int32),
        pltpu.VMEM((_CW,), jnp.int32),
        pltpu.VMEM((_CW,), jnp.int32),
        pltpu.VMEM((_CW,), jnp.int32),
        pltpu.VMEM((_CW,), jnp.int32),
        pltpu.VMEM((_CW,), jnp.int32),
        pltpu.VMEM((_CW,), jnp.int32),
        pltpu.VMEM((_CW, _H), jnp.float32),
        pltpu.VMEM((_CW, _H), jnp.float32),
        pltpu.VMEM((_CW, _H), jnp.float32),
        pltpu.VMEM_SHARED((_N, _H), jnp.float32),
        pltpu.SemaphoreType.DMA,
        pltpu.SemaphoreType.DMA,
        pltpu.SemaphoreType.DMA,
        pltpu.SemaphoreType.DMA,
        pltpu.SemaphoreType.DMA,
        pltpu.SemaphoreType.DMA,
        pltpu.SemaphoreType.DMA,
        pltpu.SemaphoreType.DMA,
        pltpu.SemaphoreType.DMA,
        pltpu.SemaphoreType.DMA,
        pltpu.SemaphoreType.DMA,
        pltpu.SemaphoreType.DMA,
        pltpu.SemaphoreType.DMA,
        pltpu.SemaphoreType.DMA,
        pltpu.SemaphoreType.DMA,
        pltpu.SemaphoreType.DMA,
    ],
)
def _scatter_kernel(y_hbm, src_hbm, dst_hbm, out_hbm,
                    ib0, ib1, ib2, db0, db1, db2, db3, db4, db5,
                    rows0, rows1, rows2, acc_sh,
                    is0, is1, is2, id0, id1, id2, id3, id4, id5,
                    g0, g1, g2, s0, s1, s2, zsem):
    c = lax.axis_index("c")
    s = lax.axis_index("s")
    wid = c * _NS + s
    nch = jnp.where(wid == _NW - 1, _RPW_LAST, _RPW)
    eoff = wid * (_RPW * _CW)

    ibs = [ib0, ib1, ib2]
    dbs = [db0, db1, db2, db3, db4, db5]
    isem = [is0, is1, is2]
    dsem = [id0, id1, id2, id3, id4, id5]
    rows = [rows0, rows1, rows2]
    gsem = [g0, g1, g2]
    ssem = [s0, s1, s2]

    def src_load(j, b):
        pltpu.async_copy(src_hbm.at[pl.ds(eoff + j * _CW, _CW)], ibs[b],
                         isem[b])

    def src_wait(j, b):
        pltpu.make_async_copy(src_hbm.at[pl.ds(eoff + j * _CW, _CW)], ibs[b],
                              isem[b]).wait()

    def dst_load(j, d):
        pltpu.async_copy(dst_hbm.at[pl.ds(eoff + j * _CW, _CW)], dbs[d],
                         dsem[d])

    def dst_wait(j, d):
        pltpu.make_async_copy(dst_hbm.at[pl.ds(eoff + j * _CW, _CW)], dbs[d],
                              dsem[d]).wait()

    def gather_start(b):
        pltpu.async_copy(y_hbm.at[ibs[b]], rows[b], gsem[b])

    def gather_wait(b):
        pltpu.make_async_copy(y_hbm.at[ibs[b]], rows[b], gsem[b]).wait()

    def scat_start(b, d):
        pltpu.async_copy(rows[b], acc_sh.at[dbs[d]], ssem[b], add=True)

    def scat_wait(b, d):
        pltpu.make_async_copy(rows[b], acc_sh.at[dbs[d]], ssem[b]).wait()

    for b in range(3):
        src_load(b, b)
    for d in range(6):
        dst_load(d, d)
    src_wait(0, 0)
    gather_start(0)
    src_wait(1, 1)
    gather_start(1)
    _zero_acc(acc_sh, rows2, zsem, s)
    plsc.subcore_barrier()

    ntrip = (nch + 2) // 3

    def trip(t, carry):
        par = t % 2
        for b in range(3):
            j = 3 * t + b

            @pl.when(j < nch)
            def _(b=b, j=j):
                gather_wait(b)
                b2 = (b + 2) % 3

                @pl.when(j >= 1)
                def _(b=b, j=j, b2=b2):
                    for q in range(2):
                        @pl.when(par == q)
                        def _(b=b, j=j, b2=b2, q=q):
                            pq = q if b != 0 else 1 - q
                            d = (b - 1) % 3 + 3 * pq
                            scat_wait(b2, d)

                            @pl.when(j + 5 < nch)
                            def _(j=j, d=d):
                                dst_load(j + 5, d)

                for q in range(2):
                    @pl.when(par == q)
                    def _(b=b, j=j, q=q):
                        d = b + 3 * q
                        dst_wait(j, d)
                        scat_start(b, d)

                @pl.when(j + 2 < nch)
                def _(b=b, j=j, b2=b2):
                    src_wait(j + 2, b2)
                    gather_start(b2)

                @pl.when(j + 3 < nch)
                def _(b=b, j=j):
                    src_load(j + 3, b)

        return carry

    lax.fori_loop(0, ntrip, trip, 0)
    scat_wait(1, 1)
    plsc.subcore_barrier()
    _write_out(acc_sh, out_hbm, c, s)


_BR = 2000
_G = _N // _BR


def _dinv_from(deg_ref):
    d = deg_ref[0, :, 0:1] + deg_ref[1, :, 0:1] + 1.0
    return lax.rsqrt(d)


def _tc_fc_body(deg_ref, x_ref, wfc_ref, bfc_ref, w1_ref, y_ref, dinv_ref):
    dinv = _dinv_from(deg_ref)
    h = jnp.maximum(
        jnp.dot(x_ref[...], wfc_ref[...], preferred_element_type=jnp.float32)
        + bfc_ref[...], 0.0)
    xw = jnp.dot(h, w1_ref[...], preferred_element_type=jnp.float32)
    y_ref[...] = xw * dinv
    dinv_ref[...] = dinv


def _tc_fc(degp, x, wfc, bfc, w1):
    return pl.pallas_call(
        _tc_fc_body,
        grid=(_G,),
        in_specs=[
            pl.BlockSpec((_NC, _BR, _H), lambda i: (0, i, 0)),
            pl.BlockSpec((_BR, _DIN), lambda i: (i, 0)),
            pl.BlockSpec((_DIN, _H), lambda i: (0, 0)),
            pl.BlockSpec((1, _H), lambda i: (0, 0)),
            pl.BlockSpec((_H, _H), lambda i: (0, 0)),
        ],
        out_specs=[pl.BlockSpec((_BR, _H), lambda i: (i, 0)),
                   pl.BlockSpec((_BR, 1), lambda i: (i, 0))],
        out_shape=[jax.ShapeDtypeStruct((_N, _H), jnp.float32),
                   jax.ShapeDtypeStruct((_N, 1), jnp.float32)],
    )(degp, x, wfc, bfc, w1)


def _tc_mid_body(dinv_ref, s_ref, y_ref, b_ref, w_ref, o_ref):
    dinv = dinv_ref[...]
    x1 = jnp.maximum(
        dinv * (s_ref[0] + s_ref[1] + y_ref[...]) + b_ref[...], 0.0)
    o_ref[...] = jnp.dot(x1, w_ref[...],
                         preferred_element_type=jnp.float32) * dinv


def _tc_mid(dinv, s_part, y, b, w):
    return pl.pallas_call(
        _tc_mid_body,
        grid=(_G,),
        in_specs=[
            pl.BlockSpec((_BR, 1), lambda i: (i, 0)),
            pl.BlockSpec((_NC, _BR, _H), lambda i: (0, i, 0)),
            pl.BlockSpec((_BR, _H), lambda i: (i, 0)),
            pl.BlockSpec((1, _H), lambda i: (0, 0)),
            pl.BlockSpec((_H, _H), lambda i: (0, 0)),
        ],
        out_specs=pl.BlockSpec((_BR, _H), lambda i: (i, 0)),
        out_shape=jax.ShapeDtypeStruct((_N, _H), jnp.float32),
    )(dinv, s_part, y, b, w)


def _tc_pool_body(dinv_ref, s_ref, y_ref, b_ref, v_ref, u_ref, wa_ref,
                  wh_ref, bh_ref, out_ref, num_acc, m_acc, den_acc):
    i = pl.program_id(0)
    dinv = dinv_ref[...]
    x2 = jnp.maximum(
        dinv * (s_ref[0] + s_ref[1] + y_ref[...]) + b_ref[...], 0.0)
    a = jnp.tanh(jnp.dot(x2, v_ref[...], preferred_element_type=jnp.float32))
    g = jax.nn.sigmoid(
        jnp.dot(x2, u_ref[...], preferred_element_type=jnp.float32))
    t = jnp.dot(a * g, wa_ref[...], preferred_element_type=jnp.float32)

    bm = jnp.max(t)
    m_old = jnp.where(i == 0, -3e38, m_acc[0])
    den_old = jnp.where(i == 0, 0.0, den_acc[0])
    num_old = jnp.where(i == 0, 0.0, num_acc[0:1, :])
    m_new = jnp.maximum(m_old, bm)
    alpha = jnp.exp(m_old - m_new)
    w = jnp.exp(t - m_new)
    num_new = num_old * alpha + lax.dot_general(
        w, x2, (((0,), (0,)), ((), ())), preferred_element_type=jnp.float32)
    den_new = den_old * alpha + jnp.sum(w)
    m_acc[0] = m_new
    den_acc[0] = den_new
    num_acc[0:1, :] = num_new
    out_ref[...] = (jnp.dot(num_new / den_new, wh_ref[...],
                            preferred_element_type=jnp.float32) + bh_ref[...])


def _tc_pool(dinv, s_part, y, b, v, u, wa, wh, bh):
    return pl.pallas_call(
        _tc_pool_body,
        grid=(_G,),
        in_specs=[
            pl.BlockSpec((_BR, 1), lambda i: (i, 0)),
            pl.BlockSpec((_NC, _BR, _H), lambda i: (0, i, 0)),
            pl.BlockSpec((_BR, _H), lambda i: (i, 0)),
            pl.BlockSpec((1, _H), lambda i: (0, 0)),
            pl.BlockSpec((_H, _H), lambda i: (0, 0)),
            pl.BlockSpec((_H, _H), lambda i: (0, 0)),
            pl.BlockSpec((_H, 1), lambda i: (0, 0)),
            pl.BlockSpec((_H, _C), lambda i: (0, 0)),
            pl.BlockSpec((1, _C), lambda i: (0, 0)),
        ],
        out_specs=pl.BlockSpec((1, _C), lambda i: (0, 0)),
        out_shape=jax.ShapeDtypeStruct((1, _C), jnp.float32),
        scratch_shapes=[
            pltpu.VMEM((8, _H), jnp.float32),
            pltpu.SMEM((1,), jnp.float32),
            pltpu.SMEM((1,), jnp.float32),
        ],
    )(dinv, s_part, y, b, v, u, wa, wh, bh)


def kernel(x, edge_index, W_fc, b_fc, W1, b1, W2, b2, V, U, w_attn,
           W_head, b_head):
    src = edge_index[0]
    dst = edge_index[1]
    dst2 = dst.reshape(_EROWS, _CW)
    degp = _deg_kernel(dst2)
    y1, dinv = _tc_fc(degp, x, W_fc, b_fc.reshape(1, _H), W1)
    s1 = _scatter_kernel(y1, src, dst)
    y2 = _tc_mid(dinv, s1, y1, b1.reshape(1, _H), W2)
    s2 = _scatter_kernel(y2, src, dst)
    out = _tc_pool(dinv, s2, y2, b2.reshape(1, _H), V, U, w_attn,
                   W_head, b_head.reshape(1, _C))
    return out


# speedup vs baseline: 1.3407x; 1.0111x over previous
import functools

import jax
import jax.numpy as jnp
from jax import lax
from jax.experimental import pallas as pl
from jax.experimental.pallas import tpu as pltpu
from jax.experimental.pallas import tpu_sc as plsc

_N = 10000
_E = 320000
_DIN = 512
_H = 128
_C = 4

_NC = 2
_NS = 16
_L = 16

_NW = _NC * _NS
_CW = 128
_EROWS = _E // _CW
_RPW = 80
_RPW_LAST = _EROWS - _RPW * (_NW - 1)
_RW = 624
_RREM = _N - _RW * _NS
_ZR = 8

_mesh = plsc.VectorSubcoreMesh(core_axis_name="c", subcore_axis_name="s")


def _fill_const(ref, nrows, vec):
    def row(i, carry):
        for j in range(_H // _L):
            ref[i, pl.ds(j * _L, _L)] = vec
        return carry

    lax.fori_loop(0, nrows, row, 0)


def _zero_acc(acc_sh, zbuf, zsem, s):
    _fill_const(zbuf, _ZR, jnp.zeros((_L,), jnp.float32))
    zero_v = zbuf.at[pl.ds(0, _ZR)]
    nz = _RW // _ZR

    def fire(i, carry):
        pltpu.async_copy(zero_v, acc_sh.at[pl.ds(s * _RW + i * _ZR, _ZR)],
                         zsem)
        return carry

    lax.fori_loop(0, nz, fire, 0)

    @pl.when(s == 0)
    def _():
        pltpu.sync_copy(zero_v, acc_sh.at[pl.ds(_RW * _NS, _ZR)])
        pltpu.sync_copy(zero_v, acc_sh.at[pl.ds(_RW * _NS + _ZR, _ZR)])

    def drain(i, carry):
        pltpu.make_async_copy(
            zero_v, acc_sh.at[pl.ds(s * _RW + i * _ZR, _ZR)], zsem).wait()
        return carry

    lax.fori_loop(0, nz, drain, 0)


def _write_out(acc_sh, out_hbm, c, s):
    pltpu.sync_copy(acc_sh.at[pl.ds(s * _RW, _RW)],
                    out_hbm.at[c, pl.ds(s * _RW, _RW)])

    @pl.when(s == 0)
    def _():
        pltpu.sync_copy(acc_sh.at[pl.ds(_RW * _NS, _RREM)],
                        out_hbm.at[c, pl.ds(_RW * _NS, _RREM)])


def _load_idx(idx2_hbm, idx_v, wid):
    @pl.when(wid < _NW - 1)
    def _():
        pltpu.sync_copy(idx2_hbm.at[pl.ds(wid * _RPW, _RPW)], idx_v)

    @pl.when(wid == _NW - 1)
    def _():
        pltpu.sync_copy(idx2_hbm.at[pl.ds((_NW - 1) * _RPW, _RPW_LAST)],
                        idx_v.at[pl.ds(0, _RPW_LAST)])


@functools.partial(
    pl.kernel,
    mesh=_mesh,
    out_type=jax.ShapeDtypeStruct((_NC, _N, _H), jnp.float32),
    scratch_types=[
        pltpu.VMEM((_RPW, _CW), jnp.int32),
        pltpu.VMEM((_CW, _H), jnp.float32),
        pltpu.VMEM((_ZR, _H), jnp.float32),
        pltpu.VMEM_SHARED((_N, _H), jnp.float32),
        pltpu.SemaphoreType.DMA,
        pltpu.SemaphoreType.DMA,
    ],
)
def _deg_kernel(dst2_hbm, out_hbm, dst_i, ones_v, zero_v, acc_sh, ssem, zsem):
    c = lax.axis_index("c")
    s = lax.axis_index("s")
    wid = c * _NS + s
    nch = jnp.where(wid == _NW - 1, _RPW_LAST, _RPW)

    _load_idx(dst2_hbm, dst_i, wid)
    _fill_const(ones_v, _CW, jnp.full((_L,), 1.0, jnp.float32))
    _zero_acc(acc_sh, zero_v, zsem, s)
    plsc.subcore_barrier()

    def fire(j, carry):
        pltpu.async_copy(ones_v, acc_sh.at[dst_i.at[j]], ssem, add=True)
        return carry

    lax.fori_loop(0, nch, fire, 0)

    def drain(j, carry):
        pltpu.make_async_copy(ones_v, acc_sh.at[dst_i.at[j]], ssem).wait()
        return carry

    lax.fori_loop(0, nch, drain, 0)
    plsc.subcore_barrier()
    _write_out(acc_sh, out_hbm, c, s)


@functools.partial(
    pl.kernel,
    mesh=_mesh,
    out_type=jax.ShapeDtypeStruct((_NC, _N, _H), jnp.float32),
    scratch_types=[
        pltpu.VMEM((_CW,), jnp.int32),
        pltpu.VMEM((_CW,), jnp.int32),
        pltpu.VMEM((_CW,), jnp.int32),
        pltpu.VMEM((_CW,), jnp.int32),
        pltpu.VMEM((_CW,), jnp.int32),
        pltpu.VMEM((_CW,), jnp.int32),
        pltpu.VMEM((_CW,), jnp.int32),
        pltpu.VMEM((_CW,), jnp.int32),
        pltpu.VMEM((_CW,), jnp.int32),
        pltpu.VMEM((_CW, _H), jnp.float32),
        pltpu.VMEM((_CW, _H), jnp.float32),
        pltpu.VMEM((_CW, _H), jnp.float32),
        pltpu.VMEM_SHARED((_N, _H), jnp.float32),
        pltpu.SemaphoreType.DMA,
        pltpu.SemaphoreType.DMA,
        pltpu.SemaphoreType.DMA,
        pltpu.SemaphoreType.DMA,
        pltpu.SemaphoreType.DMA,
        pltpu.SemaphoreType.DMA,
        pltpu.SemaphoreType.DMA,
        pltpu.SemaphoreType.DMA,
        pltpu.SemaphoreType.DMA,
        pltpu.SemaphoreType.DMA,
        pltpu.SemaphoreType.DMA,
        pltpu.SemaphoreType.DMA,
        pltpu.SemaphoreType.DMA,
        pltpu.SemaphoreType.DMA,
        pltpu.SemaphoreType.DMA,
        pltpu.SemaphoreType.DMA,
    ],
)
def _scatter_kernel(y_hbm, src_hbm, dst_hbm, out_hbm,
                    ib0, ib1, ib2, db0, db1, db2, db3, db4, db5,
                    rows0, rows1, rows2, acc_sh,
                    is0, is1, is2, id0, id1, id2, id3, id4, id5,
                    g0, g1, g2, s0, s1, s2, zsem):
    c = lax.axis_index("c")
    s = lax.axis_index("s")
    wid = c * _NS + s
    nch = jnp.where(wid == _NW - 1, _RPW_LAST, _RPW)
    eoff = wid * (_RPW * _CW)

    ibs = [ib0, ib1, ib2]
    dbs = [db0, db1, db2, db3, db4, db5]
    isem = [is0, is1, is2]
    dsem = [id0, id1, id2, id3, id4, id5]
    rows = [rows0, rows1, rows2]
    gsem = [g0, g1, g2]
    ssem = [s0, s1, s2]

    def src_load(j, b):
        pltpu.async_copy(src_hbm.at[pl.ds(eoff + j * _CW, _CW)], ibs[b],
                         isem[b])

    def src_wait(j, b):
        pltpu.make_async_copy(src_hbm.at[pl.ds(eoff + j * _CW, _CW)], ibs[b],
                              isem[b]).wait()

    def dst_load(j, d):
        pltpu.async_copy(dst_hbm.at[pl.ds(eoff + j * _CW, _CW)], dbs[d],
                         dsem[d])

    def dst_wait(j, d):
        pltpu.make_async_copy(dst_hbm.at[pl.ds(eoff + j * _CW, _CW)], dbs[d],
                              dsem[d]).wait()

    def gather_start(b):
        pltpu.async_copy(y_hbm.at[ibs[b]], rows[b], gsem[b])

    def gather_wait(b):
        pltpu.make_async_copy(y_hbm.at[ibs[b]], rows[b], gsem[b]).wait()

    def scat_start(b, d):
        pltpu.async_copy(rows[b], acc_sh.at[dbs[d]], ssem[b], add=True)

    def scat_wait(b, d):
        pltpu.make_async_copy(rows[b], acc_sh.at[dbs[d]], ssem[b]).wait()

    for b in range(3):
        src_load(b, b)
    for d in range(6):
        dst_load(d, d)
    src_wait(0, 0)
    gather_start(0)
    src_wait(1, 1)
    gather_start(1)
    _zero_acc(acc_sh, rows2, zsem, s)
    plsc.subcore_barrier()

    ntrip = (nch + 2) // 3

    def trip(t, carry):
        par = t % 2
        for b in range(3):
            j = 3 * t + b

            @pl.when(j < nch)
            def _(b=b, j=j):
                gather_wait(b)
                b2 = (b + 2) % 3

                @pl.when(j >= 1)
                def _(b=b, j=j, b2=b2):
                    for q in range(2):
                        @pl.when(par == q)
                        def _(b=b, j=j, b2=b2, q=q):
                            pq = q if b != 0 else 1 - q
                            d = (b - 1) % 3 + 3 * pq
                            scat_wait(b2, d)

                            @pl.when(j + 5 < nch)
                            def _(j=j, d=d):
                                dst_load(j + 5, d)

                for q in range(2):
                    @pl.when(par == q)
                    def _(b=b, j=j, q=q):
                        d = b + 3 * q
                        dst_wait(j, d)
                        scat_start(b, d)

                @pl.when(j + 2 < nch)
                def _(b=b, j=j, b2=b2):
                    src_wait(j + 2, b2)
                    gather_start(b2)

                @pl.when(j + 3 < nch)
                def _(b=b, j=j):
                    src_load(j + 3, b)

        return carry

    lax.fori_loop(0, ntrip, trip, 0)
    scat_wait(1, 1)
    plsc.subcore_barrier()
    _write_out(acc_sh, out_hbm, c, s)


_BR = 2000
_G = _N // _BR


def _dinv_from(deg_ref):
    d = deg_ref[0, :, 0:1] + deg_ref[1, :, 0:1] + 1.0
    return lax.rsqrt(d)


def _tc_xw_body(x_ref, wfc_ref, bfc_ref, w1_ref, xw_ref):
    h = jnp.maximum(
        jnp.dot(x_ref[...], wfc_ref[...], preferred_element_type=jnp.float32)
        + bfc_ref[...], 0.0)
    xw_ref[...] = jnp.dot(h, w1_ref[...], preferred_element_type=jnp.float32)


def _tc_xw(x, wfc, bfc, w1):
    return pl.pallas_call(
        _tc_xw_body,
        grid=(_G,),
        in_specs=[
            pl.BlockSpec((_BR, _DIN), lambda i: (i, 0)),
            pl.BlockSpec((_DIN, _H), lambda i: (0, 0)),
            pl.BlockSpec((1, _H), lambda i: (0, 0)),
            pl.BlockSpec((_H, _H), lambda i: (0, 0)),
        ],
        out_specs=pl.BlockSpec((_BR, _H), lambda i: (i, 0)),
        out_shape=jax.ShapeDtypeStruct((_N, _H), jnp.float32),
    )(x, wfc, bfc, w1)


def _tc_scale_body(deg_ref, xw_ref, y_ref, dinv_ref):
    dinv = _dinv_from(deg_ref)
    y_ref[...] = xw_ref[...] * dinv
    dinv_ref[...] = dinv


def _tc_scale(degp, xw):
    return pl.pallas_call(
        _tc_scale_body,
        grid=(_G,),
        in_specs=[
            pl.BlockSpec((_NC, _BR, _H), lambda i: (0, i, 0)),
            pl.BlockSpec((_BR, _H), lambda i: (i, 0)),
        ],
        out_specs=[pl.BlockSpec((_BR, _H), lambda i: (i, 0)),
                   pl.BlockSpec((_BR, 1), lambda i: (i, 0))],
        out_shape=[jax.ShapeDtypeStruct((_N, _H), jnp.float32),
                   jax.ShapeDtypeStruct((_N, 1), jnp.float32)],
    )(degp, xw)


def _tc_mid_body(dinv_ref, s_ref, y_ref, b_ref, w_ref, o_ref):
    dinv = dinv_ref[...]
    x1 = jnp.maximum(
        dinv * (s_ref[0] + s_ref[1] + y_ref[...]) + b_ref[...], 0.0)
    o_ref[...] = jnp.dot(x1, w_ref[...],
                         preferred_element_type=jnp.float32) * dinv


def _tc_mid(dinv, s_part, y, b, w):
    return pl.pallas_call(
        _tc_mid_body,
        grid=(_G,),
        in_specs=[
            pl.BlockSpec((_BR, 1), lambda i: (i, 0)),
            pl.BlockSpec((_NC, _BR, _H), lambda i: (0, i, 0)),
            pl.BlockSpec((_BR, _H), lambda i: (i, 0)),
            pl.BlockSpec((1, _H), lambda i: (0, 0)),
            pl.BlockSpec((_H, _H), lambda i: (0, 0)),
        ],
        out_specs=pl.BlockSpec((_BR, _H), lambda i: (i, 0)),
        out_shape=jax.ShapeDtypeStruct((_N, _H), jnp.float32),
    )(dinv, s_part, y, b, w)


def _tc_pool_body(dinv_ref, s_ref, y_ref, b_ref, v_ref, u_ref, wa_ref,
                  wh_ref, bh_ref, out_ref, num_acc, m_acc, den_acc):
    i = pl.program_id(0)
    dinv = dinv_ref[...]
    x2 = jnp.maximum(
        dinv * (s_ref[0] + s_ref[1] + y_ref[...]) + b_ref[...], 0.0)
    a = jnp.tanh(jnp.dot(x2, v_ref[...], preferred_element_type=jnp.float32))
    g = jax.nn.sigmoid(
        jnp.dot(x2, u_ref[...], preferred_element_type=jnp.float32))
    t = jnp.dot(a * g, wa_ref[...], preferred_element_type=jnp.float32)

    bm = jnp.max(t)
    m_old = jnp.where(i == 0, -3e38, m_acc[0])
    den_old = jnp.where(i == 0, 0.0, den_acc[0])
    num_old = jnp.where(i == 0, 0.0, num_acc[0:1, :])
    m_new = jnp.maximum(m_old, bm)
    alpha = jnp.exp(m_old - m_new)
    w = jnp.exp(t - m_new)
    num_new = num_old * alpha + lax.dot_general(
        w, x2, (((0,), (0,)), ((), ())), preferred_element_type=jnp.float32)
    den_new = den_old * alpha + jnp.sum(w)
    m_acc[0] = m_new
    den_acc[0] = den_new
    num_acc[0:1, :] = num_new
    out_ref[...] = (jnp.dot(num_new / den_new, wh_ref[...],
                            preferred_element_type=jnp.float32) + bh_ref[...])


def _tc_pool(dinv, s_part, y, b, v, u, wa, wh, bh):
    return pl.pallas_call(
        _tc_pool_body,
        grid=(_G,),
        in_specs=[
            pl.BlockSpec((_BR, 1), lambda i: (i, 0)),
            pl.BlockSpec((_NC, _BR, _H), lambda i: (0, i, 0)),
            pl.BlockSpec((_BR, _H), lambda i: (i, 0)),
            pl.BlockSpec((1, _H), lambda i: (0, 0)),
            pl.BlockSpec((_H, _H), lambda i: (0, 0)),
            pl.BlockSpec((_H, _H), lambda i: (0, 0)),
            pl.BlockSpec((_H, 1), lambda i: (0, 0)),
            pl.BlockSpec((_H, _C), lambda i: (0, 0)),
            pl.BlockSpec((1, _C), lambda i: (0, 0)),
        ],
        out_specs=pl.BlockSpec((1, _C), lambda i: (0, 0)),
        out_shape=jax.ShapeDtypeStruct((1, _C), jnp.float32),
        scratch_shapes=[
            pltpu.VMEM((8, _H), jnp.float32),
            pltpu.SMEM((1,), jnp.float32),
            pltpu.SMEM((1,), jnp.float32),
        ],
    )(dinv, s_part, y, b, v, u, wa, wh, bh)


def kernel(x, edge_index, W_fc, b_fc, W1, b1, W2, b2, V, U, w_attn,
           W_head, b_head):
    src = edge_index[0]
    dst = edge_index[1]
    dst2 = dst.reshape(_EROWS, _CW)
    xw1 = _tc_xw(x, W_fc, b_fc.reshape(1, _H), W1)
    degp = _deg_kernel(dst2)
    y1, dinv = _tc_scale(degp, xw1)
    s1 = _scatter_kernel(y1, src, dst)
    y2 = _tc_mid(dinv, s1, y1, b1.reshape(1, _H), W2)
    s2 = _scatter_kernel(y2, src, dst)
    out = _tc_pool(dinv, s2, y2, b2.reshape(1, _H), V, U, w_attn,
                   W_head, b_head.reshape(1, _C))
    return out
